# baseline (speedup 1.0000x reference)
import functools

import jax
import jax.numpy as jnp
from jax import lax
from jax.experimental import pallas as pl
from jax.experimental.pallas import tpu as pltpu
from jax.experimental.pallas import tpu_sc as plsc

BATCH = 4096
SEQ = 200
EMBED_DIM = 64

NUM_CORES = 2
NUM_SUBCORES = 16
NUM_WORKERS = NUM_CORES * NUM_SUBCORES

TOTAL = BATCH * SEQ
CHUNK = 128
ROWS_PER_W = TOTAL // NUM_WORKERS
N_CHUNKS = ROWS_PER_W // CHUNK

_mesh = plsc.VectorSubcoreMesh(
    core_axis_name="c", subcore_axis_name="s",
    num_cores=NUM_CORES, num_subcores=NUM_SUBCORES,
)


@functools.partial(
    pl.kernel,
    out_type=jax.ShapeDtypeStruct((TOTAL, EMBED_DIM), jnp.float32),
    mesh=_mesh,
    compiler_params=pltpu.CompilerParams(use_tc_tiling_on_sc=False),
    scratch_types=[
        pltpu.VMEM((N_CHUNKS, CHUNK), jnp.int32),
        pltpu.VMEM((CHUNK, EMBED_DIM), jnp.float32),
        pltpu.SemaphoreType.DMA,
    ],
)
def _emb_lookup(x_hbm, w_hbm, out_hbm, idx_v, rows_v, sem):
    wid = lax.axis_index("s") * NUM_CORES + lax.axis_index("c")
    pltpu.sync_copy(x_hbm.at[pl.ds(wid * N_CHUNKS, N_CHUNKS)], idx_v)
    out_base = wid * ROWS_PER_W

    def body(j, carry):
        pltpu.async_copy(w_hbm.at[idx_v.at[j]], rows_v, sem).wait()
        pltpu.sync_copy(rows_v, out_hbm.at[pl.ds(out_base + j * CHUNK, CHUNK)])
        return carry

    lax.fori_loop(0, N_CHUNKS, body, 0)


def kernel(x, W):
    x_flat = x.reshape(NUM_WORKERS * N_CHUNKS, CHUNK).astype(jnp.int32)
    out = _emb_lookup(x_flat, W)
    return out.reshape(BATCH, SEQ, EMBED_DIM)


# speedup vs baseline: 1.1134x; 1.1134x over previous
import functools

import jax
import jax.numpy as jnp
from jax import lax
from jax.experimental import pallas as pl
from jax.experimental.pallas import tpu as pltpu
from jax.experimental.pallas import tpu_sc as plsc

BATCH = 4096
SEQ = 200
EMBED_DIM = 64

NUM_CORES = 2
NUM_SUBCORES = 16
NUM_WORKERS = NUM_CORES * NUM_SUBCORES

TOTAL = BATCH * SEQ
CHUNK = 128
ROWS_PER_W = TOTAL // NUM_WORKERS
N_STEPS = ROWS_PER_W // CHUNK

NBUF = 8
DIST = 4

_mesh = plsc.VectorSubcoreMesh(
    core_axis_name="c", subcore_axis_name="s",
    num_cores=NUM_CORES, num_subcores=NUM_SUBCORES,
)


@functools.partial(
    pl.kernel,
    out_type=jax.ShapeDtypeStruct((TOTAL, EMBED_DIM), jnp.float32),
    mesh=_mesh,
    compiler_params=pltpu.CompilerParams(use_tc_tiling_on_sc=False),
    scratch_types=(
        [pltpu.VMEM((N_STEPS, CHUNK), jnp.int32),
         pltpu.VMEM((NBUF, CHUNK, EMBED_DIM), jnp.float32)]
        + [pltpu.SemaphoreType.DMA] * NBUF
        + [pltpu.SemaphoreType.DMA] * NBUF
    ),
)
def _emb_lookup(x_hbm, w_hbm, out_hbm, idx_v, rows_v, *sems):
    semg = sems[:NBUF]
    sems_ = sems[NBUF:]
    wid = lax.axis_index("s") * NUM_CORES + lax.axis_index("c")
    pltpu.sync_copy(x_hbm.at[pl.ds(wid * N_STEPS, N_STEPS)], idx_v)
    out_base = wid * ROWS_PER_W

    def fire_gather(s, b):
        pltpu.async_copy(w_hbm.at[idx_v.at[s]], rows_v.at[b], semg[b])

    def wait_gather(b):
        pltpu.make_async_copy(w_hbm.at[idx_v.at[0]], rows_v.at[b],
                              semg[b]).wait()

    def fire_scatter(s, b):
        pltpu.async_copy(rows_v.at[b],
                         out_hbm.at[pl.ds(out_base + s * CHUNK, CHUNK)],
                         sems_[b])

    def wait_scatter(b):
        pltpu.make_async_copy(rows_v.at[b],
                              out_hbm.at[pl.ds(out_base, CHUNK)],
                              sems_[b]).wait()

    def step(s, k, drain_prev):
        b = k % NBUF
        wait_gather(b)
        fire_scatter(s, b)
        nb = (k + DIST) % NBUF
        if drain_prev:
            wait_scatter(nb)
        fire_gather(s + DIST, nb)

    for s in range(DIST):
        fire_gather(s, s)
    for s in range(NBUF):
        step(s, s, drain_prev=(s >= DIST))

    def body(g, carry):
        s0 = g * NBUF
        for k in range(NBUF):
            step(s0 + k, k, drain_prev=True)
        return carry

    lax.fori_loop(1, N_STEPS // NBUF - 1, body, 0)

    for k in range(NBUF):
        s = N_STEPS - NBUF + k
        if s + DIST < N_STEPS:
            step(s, k, drain_prev=True)
        else:
            wait_gather(k)
            fire_scatter(s, k)
    for k in range(NBUF):
        wait_scatter(k)


def kernel(x, W):
    x_flat = x.reshape(NUM_WORKERS * N_STEPS, CHUNK).astype(jnp.int32)
    out = _emb_lookup(x_flat, W)
    return out.reshape(BATCH, SEQ, EMBED_DIM)
